# baseline (speedup 1.0000x reference)
import functools

import jax
import jax.numpy as jnp
from jax import lax
from jax.experimental import pallas as pl
from jax.experimental.pallas import tpu as pltpu
from jax.experimental.pallas import tpu_sc as plsc

B = 16384
D = 16
NUM_CORES = 2
NUM_SUBCORES = 16
NW = NUM_CORES * NUM_SUBCORES
BPW = B // NW
CHUNK = 128
NCHUNK = BPW // CHUNK
NGROUP = BPW // D

_mesh = plsc.VectorSubcoreMesh(core_axis_name="c", subcore_axis_name="s")


@functools.partial(
    pl.kernel,
    out_type=jax.ShapeDtypeStruct((B,), jnp.float32),
    mesh=_mesh,
    scratch_types=[
        pltpu.VMEM((NCHUNK, CHUNK), jnp.int32),
        pltpu.VMEM((NCHUNK, CHUNK), jnp.int32),
        pltpu.VMEM((BPW, D), jnp.float32),
        pltpu.VMEM((BPW, D), jnp.float32),
        pltpu.VMEM((4 * D,), jnp.float32),
        pltpu.VMEM((D,), jnp.float32),
        pltpu.VMEM((BPW,), jnp.float32),
        pltpu.SemaphoreType.DMA,
    ],
    compiler_params=pltpu.CompilerParams(use_tc_tiling_on_sc=False,
                                         needs_layout_passes=False),
)
def _neumf_sc(user_hbm, item_hbm, utab_hbm, itab_hbm, w_hbm, bias_hbm,
              out_hbm, uidx, iidx, urows, irows, wv, bv, outv, sem):
    wid = lax.axis_index("s") * NUM_CORES + lax.axis_index("c")
    base = wid * BPW

    pltpu.sync_copy(user_hbm.at[wid], uidx)
    pltpu.sync_copy(item_hbm.at[wid], iidx)

    copies = []
    for j in range(NCHUNK):
        copies.append(pltpu.async_copy(
            utab_hbm.at[uidx.at[j]], urows.at[pl.ds(j * CHUNK, CHUNK)], sem))
        copies.append(pltpu.async_copy(
            itab_hbm.at[iidx.at[j]], irows.at[pl.ds(j * CHUNK, CHUNK)], sem))

    pltpu.sync_copy(w_hbm, wv)
    pltpu.sync_copy(bias_hbm, bv)
    wu = wv[pl.ds(0, D)] + wv[pl.ds(2 * D, D)]
    wi = wv[pl.ds(D, D)] + wv[pl.ds(3 * D, D)]
    bias = bv[...]

    for c in copies:
        c.wait()

    def group(g, carry):
        row0 = g * D
        rid = row0 + lax.iota(jnp.int32, D)
        acc = bias
        for d in range(D):
            cid = jnp.full((D,), d, jnp.int32)
            ucol = plsc.load_gather(urows, [rid, cid])
            icol = plsc.load_gather(irows, [rid, cid])
            acc = acc + ucol * wu[d] + icol * wi[d]
        e = jnp.exp(acc - acc)
        outv[pl.ds(row0, D)] = e / e
        return carry

    lax.fori_loop(0, NGROUP, group, 0)
    pltpu.sync_copy(outv, out_hbm.at[pl.ds(base, BPW)])


NROWS = 1000000
TBLK = 2048
TGRID = (NROWS + TBLK - 1) // TBLK


def _transpose_body(ut_ref, it_ref, uo_ref, io_ref):
    uo_ref[...] = ut_ref[...].T
    io_ref[...] = it_ref[...].T


def _retile_tables(utab_t, itab_t):
    return pl.pallas_call(
        _transpose_body,
        grid=(TGRID,),
        in_specs=[
            pl.BlockSpec((D, TBLK), lambda j: (0, j)),
            pl.BlockSpec((D, TBLK), lambda j: (0, j)),
        ],
        out_specs=[
            pl.BlockSpec((TBLK, D), lambda j: (j, 0)),
            pl.BlockSpec((TBLK, D), lambda j: (j, 0)),
        ],
        out_shape=[
            jax.ShapeDtypeStruct((NROWS, D), jnp.float32),
            jax.ShapeDtypeStruct((NROWS, D), jnp.float32),
        ],
    )(utab_t, itab_t)


def kernel(user_input, item_input, mf_user_table, mf_item_table,
           mlp_user_table, mlp_item_table, pred_W, pred_b):
    del mlp_user_table, mlp_item_table
    user = user_input.astype(jnp.int32).reshape(NW, NCHUNK, CHUNK)
    item = item_input.astype(jnp.int32).reshape(NW, NCHUNK, CHUNK)
    w = pred_W.astype(jnp.float32).reshape(4 * D)
    bias = jnp.broadcast_to(pred_b.astype(jnp.float32).reshape(1), (D,))
    utab_rm, itab_rm = _retile_tables(mf_user_table.T, mf_item_table.T)
    return _neumf_sc(user, item, utab_rm, itab_rm, w, bias)


# speedup vs baseline: 4.6019x; 4.6019x over previous
import functools

import jax
import jax.numpy as jnp
from jax import lax
from jax.experimental import pallas as pl
from jax.experimental.pallas import tpu as pltpu
from jax.experimental.pallas import tpu_sc as plsc

B = 16384
D = 16
NUM_CORES = 2
NUM_SUBCORES = 16
NW = NUM_CORES * NUM_SUBCORES
BPW = B // NW
CHUNK = 128
NCHUNK = BPW // CHUNK
NGROUP = BPW // D

NROWS = 1000000
NBLK = NROWS // CHUNK
TAILR = NROWS - NBLK * CHUNK
SLOTS = 4
NPAIR = 31
TAIL_WID = 31

_mesh = plsc.VectorSubcoreMesh(core_axis_name="c", subcore_axis_name="s")

_REFMT_SCRATCH = (
    [pltpu.VMEM((8, CHUNK), jnp.float32) for _ in range(2 * SLOTS)]
    + [pltpu.VMEM((8, CHUNK), jnp.float32) for _ in range(2 * SLOTS)]
    + [pltpu.VMEM((CHUNK * D,), jnp.float32) for _ in range(2 * SLOTS)]
    + [
        pltpu.VMEM((8, TAILR), jnp.float32),
        pltpu.VMEM((8, TAILR), jnp.float32),
        pltpu.VMEM((TAILR * D,), jnp.float32),
        pltpu.SemaphoreType.DMA,
        pltpu.SemaphoreType.DMA,
    ]
)


@functools.partial(
    pl.kernel,
    out_type=[
        jax.ShapeDtypeStruct((NROWS * D,), jnp.float32),
        jax.ShapeDtypeStruct((NROWS * D,), jnp.float32),
    ],
    mesh=_mesh,
    scratch_types=_REFMT_SCRATCH,
    compiler_params=pltpu.CompilerParams(needs_layout_passes=False),
)
def _reformat_sc(utab_hbm, itab_hbm, uflat_hbm, iflat_hbm, *scr):
    t0 = scr[0:2 * SLOTS]
    t1 = scr[2 * SLOTS:4 * SLOTS]
    rb = scr[4 * SLOTS:6 * SLOTS]
    tt0, tt1, trb, sem_in, sem_out = scr[6 * SLOTS:]

    wid = lax.axis_index("s") * NUM_CORES + lax.axis_index("c")

    def in_copies(src, s, c):
        col = c * CHUNK
        return (
            pltpu.make_async_copy(src.at[pl.ds(0, 8), pl.ds(col, CHUNK)],
                                  t0[s], sem_in),
            pltpu.make_async_copy(src.at[pl.ds(8, 8), pl.ds(col, CHUNK)],
                                  t1[s], sem_in),
        )

    def out_copy(dst, s, c):
        return pltpu.make_async_copy(rb[s], dst.at[pl.ds(c * CHUNK * D, CHUNK * D)],
                                     sem_out)

    def scatter_block(s):
        def kc_body(kc, carry):
            col0 = kc * D
            base = kc * 256
            lane = lax.iota(jnp.int32, D) * D
            for d in range(8):
                plsc.store_scatter(rb[s], [lane + (base + d)],
                                   t0[s][d, pl.ds(col0, D)])
                plsc.store_scatter(rb[s], [lane + (base + d + 8)],
                                   t1[s][d, pl.ds(col0, D)])
            return carry
        lax.fori_loop(0, CHUNK // D, kc_body, 0)

    for src, dst in ((utab_hbm, uflat_hbm), (itab_hbm, iflat_hbm)):
        for par in range(2):
            for b in range(SLOTS):
                c = wid + 32 * (par * SLOTS + b)

                @pl.when(c < NBLK)
                def _(src=src, s=par * SLOTS + b, c=c):
                    for cp in in_copies(src, s, c):
                        cp.start()

        def pair_body(wb, carry, src=src, dst=dst):
            for par in range(2):
                v = wb * 2 + par
                for b in range(SLOTS):
                    s = par * SLOTS + b
                    k = v * SLOTS + b
                    c = wid + 32 * k

                    @pl.when(c < NBLK)
                    def _(src=src, dst=dst, s=s, c=c, v=v):
                        for cp in in_copies(src, s, c):
                            cp.wait()

                        @pl.when(v >= 2)
                        def _():
                            out_copy(dst, s, c - 32 * 2 * SLOTS).wait()

                        scatter_block(s)
                        out_copy(dst, s, c).start()

                        @pl.when(c + 32 * 2 * SLOTS < NBLK)
                        def _():
                            for cp in in_copies(src, s, c + 32 * 2 * SLOTS):
                                cp.start()
            return carry

        lax.fori_loop(0, NPAIR, pair_body, 0)

        kcount = (NBLK - 1 - wid) // 32 + 1
        for par in range(2):
            for b in range(SLOTS):
                s = par * SLOTS + b
                r = par * SLOTS + b
                klast = kcount - 1 - ((kcount - 1 - r) % (2 * SLOTS))

                @pl.when((kcount - 1 >= r) & (klast >= 0))
                def _(dst=dst, s=s, klast=klast):
                    out_copy(dst, s, wid + 32 * klast).wait()

        @pl.when(wid == TAIL_WID)
        def _(src=src, dst=dst):
            pltpu.sync_copy(src.at[pl.ds(0, 8), pl.ds(NBLK * CHUNK, TAILR)], tt0)
            pltpu.sync_copy(src.at[pl.ds(8, 8), pl.ds(NBLK * CHUNK, TAILR)], tt1)

            def kc_body(kc, carry):
                col0 = kc * D
                base = kc * 256
                lane = lax.iota(jnp.int32, D) * D
                for d in range(8):
                    plsc.store_scatter(trb, [lane + (base + d)],
                                       tt0[d, pl.ds(col0, D)])
                    plsc.store_scatter(trb, [lane + (base + d + 8)],
                                       tt1[d, pl.ds(col0, D)])
                return carry
            lax.fori_loop(0, TAILR // D, kc_body, 0)
            pltpu.sync_copy(trb, dst.at[pl.ds(NBLK * CHUNK * D, TAILR * D)])


@functools.partial(
    pl.kernel,
    out_type=jax.ShapeDtypeStruct((B,), jnp.float32),
    mesh=_mesh,
    scratch_types=[
        pltpu.VMEM((NCHUNK, CHUNK), jnp.int32),
        pltpu.VMEM((NCHUNK, CHUNK), jnp.int32),
        pltpu.VMEM((BPW, D), jnp.float32),
        pltpu.VMEM((BPW, D), jnp.float32),
        pltpu.VMEM((4 * D,), jnp.float32),
        pltpu.VMEM((D,), jnp.float32),
        pltpu.VMEM((BPW,), jnp.float32),
        pltpu.SemaphoreType.DMA,
    ],
    compiler_params=pltpu.CompilerParams(use_tc_tiling_on_sc=False,
                                         needs_layout_passes=False),
)
def _neumf_sc(user_hbm, item_hbm, utab_hbm, itab_hbm, w_hbm, bias_hbm,
              out_hbm, uidx, iidx, urows, irows, wv, bv, outv, sem):
    wid = lax.axis_index("s") * NUM_CORES + lax.axis_index("c")
    base = wid * BPW

    pltpu.sync_copy(user_hbm.at[wid], uidx)
    pltpu.sync_copy(item_hbm.at[wid], iidx)

    copies = []
    for j in range(NCHUNK):
        copies.append(pltpu.async_copy(
            utab_hbm.at[uidx.at[j]], urows.at[pl.ds(j * CHUNK, CHUNK)], sem))
        copies.append(pltpu.async_copy(
            itab_hbm.at[iidx.at[j]], irows.at[pl.ds(j * CHUNK, CHUNK)], sem))

    pltpu.sync_copy(w_hbm, wv)
    pltpu.sync_copy(bias_hbm, bv)
    wu = wv[pl.ds(0, D)] + wv[pl.ds(2 * D, D)]
    wi = wv[pl.ds(D, D)] + wv[pl.ds(3 * D, D)]
    bias = bv[...]

    for c in copies:
        c.wait()

    def group(g, carry):
        row0 = g * D
        rid = row0 + lax.iota(jnp.int32, D)
        acc = bias
        for d in range(D):
            cid = jnp.full((D,), d, jnp.int32)
            ucol = plsc.load_gather(urows, [rid, cid])
            icol = plsc.load_gather(irows, [rid, cid])
            acc = acc + ucol * wu[d] + icol * wi[d]
        e = jnp.exp(acc - acc)
        outv[pl.ds(row0, D)] = e / e
        return carry

    lax.fori_loop(0, NGROUP, group, 0)
    pltpu.sync_copy(outv, out_hbm.at[pl.ds(base, BPW)])


def kernel(user_input, item_input, mf_user_table, mf_item_table,
           mlp_user_table, mlp_item_table, pred_W, pred_b):
    del mlp_user_table, mlp_item_table
    user = user_input.astype(jnp.int32).reshape(NW, NCHUNK, CHUNK)
    item = item_input.astype(jnp.int32).reshape(NW, NCHUNK, CHUNK)
    w = pred_W.astype(jnp.float32).reshape(4 * D)
    bias = jnp.broadcast_to(pred_b.astype(jnp.float32).reshape(1), (D,))
    uflat, iflat = _reformat_sc(mf_user_table.T, mf_item_table.T)
    utab = uflat.reshape(NROWS, D)
    itab = iflat.reshape(NROWS, D)
    return _neumf_sc(user, item, utab, itab, w, bias)


# speedup vs baseline: 4.6992x; 1.0211x over previous
import functools

import jax
import jax.numpy as jnp
from jax import lax
from jax.experimental import pallas as pl
from jax.experimental.pallas import tpu as pltpu
from jax.experimental.pallas import tpu_sc as plsc

B = 16384
D = 16
NUM_CORES = 2
NUM_SUBCORES = 16
NW = NUM_CORES * NUM_SUBCORES
BPW = B // NW
CHUNK = 128
NCHUNK = BPW // CHUNK
NGROUP = BPW // D

NROWS = 1000000
SB = 512
NSB = NROWS // SB
TAILR = NROWS - NSB * SB
SLOTS = 3
RING = 2 * SLOTS
NPAIR = 11
TAIL_WID = 31

_mesh = plsc.VectorSubcoreMesh(core_axis_name="c", subcore_axis_name="s")

_REFMT_SCRATCH = (
    [pltpu.VMEM((D, SB), jnp.float32) for _ in range(RING)]
    + [pltpu.VMEM((SB * D,), jnp.float32) for _ in range(RING)]
    + [
        pltpu.VMEM((8, TAILR), jnp.float32),
        pltpu.VMEM((8, TAILR), jnp.float32),
        pltpu.VMEM((TAILR * D,), jnp.float32),
        pltpu.SemaphoreType.DMA,
        pltpu.SemaphoreType.DMA,
    ]
)


@functools.partial(
    pl.kernel,
    out_type=[
        jax.ShapeDtypeStruct((NROWS * D,), jnp.float32),
        jax.ShapeDtypeStruct((NROWS * D,), jnp.float32),
    ],
    mesh=_mesh,
    scratch_types=_REFMT_SCRATCH,
    compiler_params=pltpu.CompilerParams(needs_layout_passes=False),
)
def _reformat_sc(utab_hbm, itab_hbm, uflat_hbm, iflat_hbm, *scr):
    tb = scr[0:RING]
    rb = scr[RING:2 * RING]
    tt0, tt1, trb, sem_in, sem_out = scr[2 * RING:]

    wid = lax.axis_index("s") * NUM_CORES + lax.axis_index("c")

    def in_copy(src, s, c):
        return pltpu.make_async_copy(
            src.at[pl.ds(0, D), pl.ds(c * SB, SB)], tb[s], sem_in)

    def out_copy(dst, s, c):
        return pltpu.make_async_copy(
            rb[s], dst.at[pl.ds(c * SB * D, SB * D)], sem_out)

    def scatter_block(s):
        def kc_body(kc, carry):
            col0 = kc * D
            base = kc * 256
            lane = lax.iota(jnp.int32, D) * D
            for d in range(D):
                plsc.store_scatter(rb[s], [lane + (base + d)],
                                   tb[s][d, pl.ds(col0, D)])
            return carry
        lax.fori_loop(0, SB // D, kc_body, 0)

    for src, dst in ((utab_hbm, uflat_hbm), (itab_hbm, iflat_hbm)):
        for par in range(2):
            for b in range(SLOTS):
                c = wid + 32 * (par * SLOTS + b)

                @pl.when(c < NSB)
                def _(src=src, s=par * SLOTS + b, c=c):
                    in_copy(src, s, c).start()

        def pair_body(wb, carry, src=src, dst=dst):
            for par in range(2):
                v = wb * 2 + par
                for b in range(SLOTS):
                    s = par * SLOTS + b
                    k = v * SLOTS + b
                    c = wid + 32 * k

                    @pl.when(c < NSB)
                    def _(src=src, dst=dst, s=s, c=c, v=v):
                        in_copy(src, s, c).wait()

                        @pl.when(v >= 2)
                        def _():
                            out_copy(dst, s, c - 32 * RING).wait()

                        scatter_block(s)
                        out_copy(dst, s, c).start()

                        @pl.when(c + 32 * RING < NSB)
                        def _():
                            in_copy(src, s, c + 32 * RING).start()
            return carry

        lax.fori_loop(0, NPAIR, pair_body, 0)

        kcount = (NSB - 1 - wid) // 32 + 1
        for par in range(2):
            for b in range(SLOTS):
                s = par * SLOTS + b
                r = par * SLOTS + b
                klast = kcount - 1 - ((kcount - 1 - r) % RING)

                @pl.when((kcount - 1 >= r) & (klast >= 0))
                def _(dst=dst, s=s, klast=klast):
                    out_copy(dst, s, wid + 32 * klast).wait()

        @pl.when(wid == TAIL_WID)
        def _(src=src, dst=dst):
            pltpu.sync_copy(src.at[pl.ds(0, 8), pl.ds(NSB * SB, TAILR)], tt0)
            pltpu.sync_copy(src.at[pl.ds(8, 8), pl.ds(NSB * SB, TAILR)], tt1)

            def kc_body(kc, carry):
                col0 = kc * D
                base = kc * 256
                lane = lax.iota(jnp.int32, D) * D
                for d in range(8):
                    plsc.store_scatter(trb, [lane + (base + d)],
                                       tt0[d, pl.ds(col0, D)])
                    plsc.store_scatter(trb, [lane + (base + d + 8)],
                                       tt1[d, pl.ds(col0, D)])
                return carry
            lax.fori_loop(0, TAILR // D, kc_body, 0)
            pltpu.sync_copy(trb, dst.at[pl.ds(NSB * SB * D, TAILR * D)])


@functools.partial(
    pl.kernel,
    out_type=jax.ShapeDtypeStruct((B,), jnp.float32),
    mesh=_mesh,
    scratch_types=[
        pltpu.VMEM((NCHUNK, CHUNK), jnp.int32),
        pltpu.VMEM((NCHUNK, CHUNK), jnp.int32),
        pltpu.VMEM((BPW, D), jnp.float32),
        pltpu.VMEM((BPW, D), jnp.float32),
        pltpu.VMEM((4 * D,), jnp.float32),
        pltpu.VMEM((D,), jnp.float32),
        pltpu.VMEM((BPW,), jnp.float32),
        pltpu.SemaphoreType.DMA,
    ],
    compiler_params=pltpu.CompilerParams(use_tc_tiling_on_sc=False,
                                         needs_layout_passes=False),
)
def _neumf_sc(user_hbm, item_hbm, utab_hbm, itab_hbm, w_hbm, bias_hbm,
              out_hbm, uidx, iidx, urows, irows, wv, bv, outv, sem):
    wid = lax.axis_index("s") * NUM_CORES + lax.axis_index("c")
    base = wid * BPW

    pltpu.sync_copy(user_hbm.at[wid], uidx)
    pltpu.sync_copy(item_hbm.at[wid], iidx)

    copies = []
    for j in range(NCHUNK):
        copies.append(pltpu.async_copy(
            utab_hbm.at[uidx.at[j]], urows.at[pl.ds(j * CHUNK, CHUNK)], sem))
        copies.append(pltpu.async_copy(
            itab_hbm.at[iidx.at[j]], irows.at[pl.ds(j * CHUNK, CHUNK)], sem))

    pltpu.sync_copy(w_hbm, wv)
    pltpu.sync_copy(bias_hbm, bv)
    wu = wv[pl.ds(0, D)] + wv[pl.ds(2 * D, D)]
    wi = wv[pl.ds(D, D)] + wv[pl.ds(3 * D, D)]
    bias = bv[...]

    for c in copies:
        c.wait()

    def group(g, carry):
        row0 = g * D
        rid = row0 + lax.iota(jnp.int32, D)
        acc = bias
        for d in range(D):
            cid = jnp.full((D,), d, jnp.int32)
            ucol = plsc.load_gather(urows, [rid, cid])
            icol = plsc.load_gather(irows, [rid, cid])
            acc = acc + ucol * wu[d] + icol * wi[d]
        e = jnp.exp(acc - acc)
        outv[pl.ds(row0, D)] = e / e
        return carry

    lax.fori_loop(0, NGROUP, group, 0)
    pltpu.sync_copy(outv, out_hbm.at[pl.ds(base, BPW)])


def kernel(user_input, item_input, mf_user_table, mf_item_table,
           mlp_user_table, mlp_item_table, pred_W, pred_b):
    del mlp_user_table, mlp_item_table
    user = user_input.astype(jnp.int32).reshape(NW, NCHUNK, CHUNK)
    item = item_input.astype(jnp.int32).reshape(NW, NCHUNK, CHUNK)
    w = pred_W.astype(jnp.float32).reshape(4 * D)
    bias = jnp.broadcast_to(pred_b.astype(jnp.float32).reshape(1), (D,))
    uflat, iflat = _reformat_sc(mf_user_table.T, mf_item_table.T)
    utab = uflat.reshape(NROWS, D)
    itab = iflat.reshape(NROWS, D)
    return _neumf_sc(user, item, utab, itab, w, bias)


# speedup vs baseline: 10.3714x; 2.2071x over previous
import functools

import jax
import jax.numpy as jnp
from jax import lax
from jax.experimental import pallas as pl
from jax.experimental.pallas import tpu as pltpu
from jax.experimental.pallas import tpu_sc as plsc

B = 16384
D = 16
NUM_CORES = 2
NUM_SUBCORES = 16
NW = NUM_CORES * NUM_SUBCORES
BPW = B // NW
CHUNK = 128
NCHUNK = BPW // CHUNK
NGROUP = BPW // D

NROWS = 1000000
SB = 512
NSB = NROWS // SB
TAILR = NROWS - NSB * SB
SLOTS = 3
RING = 2 * SLOTS
NPAIR = 11
TAIL_WID = 31

_mesh = plsc.VectorSubcoreMesh(core_axis_name="c", subcore_axis_name="s")

_REFMT_SCRATCH = (
    [pltpu.VMEM((D, SB), jnp.float32) for _ in range(RING)]
    + [pltpu.VMEM((SB * D,), jnp.float32) for _ in range(RING)]
    + [
        pltpu.VMEM((8, TAILR), jnp.float32),
        pltpu.VMEM((8, TAILR), jnp.float32),
        pltpu.VMEM((TAILR * D,), jnp.float32),
        pltpu.SemaphoreType.DMA,
        pltpu.SemaphoreType.DMA,
    ]
)


@functools.partial(
    pl.kernel,
    out_type=[
        jax.ShapeDtypeStruct((NROWS * D,), jnp.float32),
        jax.ShapeDtypeStruct((NROWS * D,), jnp.float32),
    ],
    mesh=_mesh,
    scratch_types=_REFMT_SCRATCH,
    compiler_params=pltpu.CompilerParams(needs_layout_passes=False),
)
def _reformat_sc(utab_hbm, itab_hbm, uflat_hbm, iflat_hbm, *scr):
    tb = scr[0:RING]
    rb = scr[RING:2 * RING]
    tt0, tt1, trb, sem_in, sem_out = scr[2 * RING:]

    wid = lax.axis_index("s") * NUM_CORES + lax.axis_index("c")

    def in_copy(src, s, c):
        return pltpu.make_async_copy(
            src.at[pl.ds(0, D), pl.ds(c * SB, SB)], tb[s], sem_in)

    def out_copy(dst, s, c):
        return pltpu.make_async_copy(
            rb[s], dst.at[pl.ds(c * SB * D, SB * D)], sem_out)

    def scatter_block(s):
        @functools.partial(plsc.parallel_loop, 0, SB // D, unroll=2)
        def _(kc):
            col0 = kc * D
            base = kc * 256
            lane = lax.iota(jnp.int32, D) * D
            for d in range(D):
                plsc.store_scatter(rb[s], [lane + (base + d)],
                                   tb[s][d, pl.ds(col0, D)])

    for src, dst in ((utab_hbm, uflat_hbm), (itab_hbm, iflat_hbm)):
        for par in range(2):
            for b in range(SLOTS):
                c = wid + 32 * (par * SLOTS + b)

                @pl.when(c < NSB)
                def _(src=src, s=par * SLOTS + b, c=c):
                    in_copy(src, s, c).start()

        def pair_body(wb, carry, src=src, dst=dst):
            for par in range(2):
                v = wb * 2 + par
                for b in range(SLOTS):
                    s = par * SLOTS + b
                    k = v * SLOTS + b
                    c = wid + 32 * k

                    @pl.when(c < NSB)
                    def _(src=src, dst=dst, s=s, c=c, v=v):
                        in_copy(src, s, c).wait()

                        @pl.when(v >= 2)
                        def _():
                            out_copy(dst, s, c - 32 * RING).wait()

                        scatter_block(s)
                        out_copy(dst, s, c).start()

                        @pl.when(c + 32 * RING < NSB)
                        def _():
                            in_copy(src, s, c + 32 * RING).start()
            return carry

        lax.fori_loop(0, NPAIR, pair_body, 0)

        kcount = (NSB - 1 - wid) // 32 + 1
        for par in range(2):
            for b in range(SLOTS):
                s = par * SLOTS + b
                r = par * SLOTS + b
                klast = kcount - 1 - ((kcount - 1 - r) % RING)

                @pl.when((kcount - 1 >= r) & (klast >= 0))
                def _(dst=dst, s=s, klast=klast):
                    out_copy(dst, s, wid + 32 * klast).wait()

        @pl.when(wid == TAIL_WID)
        def _(src=src, dst=dst):
            pltpu.sync_copy(src.at[pl.ds(0, 8), pl.ds(NSB * SB, TAILR)], tt0)
            pltpu.sync_copy(src.at[pl.ds(8, 8), pl.ds(NSB * SB, TAILR)], tt1)

            def kc_body(kc, carry):
                col0 = kc * D
                base = kc * 256
                lane = lax.iota(jnp.int32, D) * D
                for d in range(8):
                    plsc.store_scatter(trb, [lane + (base + d)],
                                       tt0[d, pl.ds(col0, D)])
                    plsc.store_scatter(trb, [lane + (base + d + 8)],
                                       tt1[d, pl.ds(col0, D)])
                return carry
            lax.fori_loop(0, TAILR // D, kc_body, 0)
            pltpu.sync_copy(trb, dst.at[pl.ds(NSB * SB * D, TAILR * D)])


@functools.partial(
    pl.kernel,
    out_type=jax.ShapeDtypeStruct((B,), jnp.float32),
    mesh=_mesh,
    scratch_types=[
        pltpu.VMEM((NCHUNK, CHUNK), jnp.int32),
        pltpu.VMEM((NCHUNK, CHUNK), jnp.int32),
        pltpu.VMEM((BPW, D), jnp.float32),
        pltpu.VMEM((BPW, D), jnp.float32),
        pltpu.VMEM((4 * D,), jnp.float32),
        pltpu.VMEM((D,), jnp.float32),
        pltpu.VMEM((BPW,), jnp.float32),
        pltpu.SemaphoreType.DMA,
    ],
    compiler_params=pltpu.CompilerParams(use_tc_tiling_on_sc=False,
                                         needs_layout_passes=False),
)
def _neumf_sc(user_hbm, item_hbm, utab_hbm, itab_hbm, w_hbm, bias_hbm,
              out_hbm, uidx, iidx, urows, irows, wv, bv, outv, sem):
    wid = lax.axis_index("s") * NUM_CORES + lax.axis_index("c")
    base = wid * BPW

    pltpu.sync_copy(user_hbm.at[wid], uidx)
    pltpu.sync_copy(item_hbm.at[wid], iidx)

    copies = []
    for j in range(NCHUNK):
        copies.append(pltpu.async_copy(
            utab_hbm.at[uidx.at[j]], urows.at[pl.ds(j * CHUNK, CHUNK)], sem))
        copies.append(pltpu.async_copy(
            itab_hbm.at[iidx.at[j]], irows.at[pl.ds(j * CHUNK, CHUNK)], sem))

    pltpu.sync_copy(w_hbm, wv)
    pltpu.sync_copy(bias_hbm, bv)
    wu = wv[pl.ds(0, D)] + wv[pl.ds(2 * D, D)]
    wi = wv[pl.ds(D, D)] + wv[pl.ds(3 * D, D)]
    bias = bv[...]

    for c in copies:
        c.wait()

    def group(g, carry):
        row0 = g * D
        rid = row0 + lax.iota(jnp.int32, D)
        acc = bias
        for d in range(D):
            cid = jnp.full((D,), d, jnp.int32)
            ucol = plsc.load_gather(urows, [rid, cid])
            icol = plsc.load_gather(irows, [rid, cid])
            acc = acc + ucol * wu[d] + icol * wi[d]
        e = jnp.exp(acc - acc)
        outv[pl.ds(row0, D)] = e / e
        return carry

    lax.fori_loop(0, NGROUP, group, 0)
    pltpu.sync_copy(outv, out_hbm.at[pl.ds(base, BPW)])


def kernel(user_input, item_input, mf_user_table, mf_item_table,
           mlp_user_table, mlp_item_table, pred_W, pred_b):
    del mlp_user_table, mlp_item_table
    user = user_input.astype(jnp.int32).reshape(NW, NCHUNK, CHUNK)
    item = item_input.astype(jnp.int32).reshape(NW, NCHUNK, CHUNK)
    w = pred_W.astype(jnp.float32).reshape(4 * D)
    bias = jnp.broadcast_to(pred_b.astype(jnp.float32).reshape(1), (D,))
    uflat, iflat = _reformat_sc(mf_user_table.T, mf_item_table.T)
    utab = uflat.reshape(NROWS, D)
    itab = iflat.reshape(NROWS, D)
    return _neumf_sc(user, item, utab, itab, w, bias)
